# baseline (speedup 1.0000x reference)
import jax
import jax.numpy as jnp
from jax.experimental import pallas as pl

F = 10000
H = 64
BF = 768
NB = -(-F // BF)
FPAD = NB * BF


def _body(m_ref, wc_ref, ht_ref, x_ref, wx_ref, bias_ref,
          w1_ref, b1_ref, w2_ref, b2_ref, accz_ref, accm_ref, out_ref):
    k = pl.program_id(0)

    @pl.when(k == 0)
    def _():
        accz_ref[...] = jnp.zeros_like(accz_ref)
        accm_ref[...] = jnp.zeros_like(accm_ref)

    m = m_ref[...]
    wc = wc_ref[...]
    ht = ht_ref[...]
    h = jnp.sum(wc * ht[None, :, :], axis=1)
    z = x_ref[...] * wx_ref[...] + bias_ref[...] + h
    zs = jnp.where(m > 0.0, z, 0.0)
    accz_ref[...] += m * jnp.tanh(zs)
    accm_ref[...] += jnp.broadcast_to(m, (8, BF))

    @pl.when(k == NB - 1)
    def _():
        s = jnp.sum(accz_ref[...], axis=1)
        cnt = jnp.sum(accm_ref[0:1, :])
        iv = (s / jnp.maximum(cnt, 1.0)).reshape(1, H)
        hmlp = jnp.maximum(
            jax.lax.dot_general(iv, w1_ref[...],
                                (((1,), (1,)), ((), ()))) + b1_ref[...],
            0.0)
        logits = jax.lax.dot_general(hmlp, w2_ref[...],
                                     (((1,), (1,)), ((), ()))) + b2_ref[...]
        mx = jnp.max(logits, axis=1, keepdims=True)
        e = jnp.exp(logits - mx)
        p = e / jnp.sum(e, axis=1, keepdims=True)
        pad = jnp.concatenate([p, jnp.zeros((1, 126), jnp.float32)], axis=1)
        out_ref[...] = jnp.broadcast_to(pad, (8, 128))


@jax.jit
def _run(X, mask, Ht, Wx, Wc, rnn_bias, W1, b1, W2, b2):
    wcT = jnp.transpose(Wc, (1, 2, 0))
    htT = Ht.T
    wxT = Wx[:, :, 0].T
    biasT = rnn_bias.T
    xb = X.reshape(1, F)
    maskp = jnp.pad(mask.astype(jnp.float32), (0, FPAD - F)).reshape(1, FPAD)

    _, _, res = pl.pallas_call(
        _body,
        grid=(NB,),
        in_specs=[
            pl.BlockSpec((1, BF), lambda k: (0, k)),
            pl.BlockSpec((H, H, BF), lambda k: (0, 0, k)),
            pl.BlockSpec((H, BF), lambda k: (0, k)),
            pl.BlockSpec((1, BF), lambda k: (0, k)),
            pl.BlockSpec((H, BF), lambda k: (0, k)),
            pl.BlockSpec((H, BF), lambda k: (0, k)),
            pl.BlockSpec((H, H), lambda k: (0, 0)),
            pl.BlockSpec((1, H), lambda k: (0, 0)),
            pl.BlockSpec((2, H), lambda k: (0, 0)),
            pl.BlockSpec((1, 2), lambda k: (0, 0)),
        ],
        out_specs=[
            pl.BlockSpec((H, BF), lambda k: (0, 0)),
            pl.BlockSpec((8, BF), lambda k: (0, 0)),
            pl.BlockSpec((8, 128), lambda k: (0, 0)),
        ],
        out_shape=[
            jax.ShapeDtypeStruct((H, BF), jnp.float32),
            jax.ShapeDtypeStruct((8, BF), jnp.float32),
            jax.ShapeDtypeStruct((8, 128), jnp.float32),
        ],
    )(maskp, wcT, htT, xb, wxT, biasT,
      W1, b1.reshape(1, H), W2, b2.reshape(1, 2))
    return res[0, :2]


def kernel(tim, X, X_hap, mask, Ht, Wx, Wc, rnn_bias, W1, b1, W2, b2):
    pred = _run(X, mask, Ht, Wx, Wc, rnn_bias, W1, b1, W2, b2)
    H_curr = jnp.zeros((F, H), dtype=jnp.float32)
    return pred, H_curr


# speedup vs baseline: 1.0205x; 1.0205x over previous
import jax
import jax.numpy as jnp
from jax.experimental import pallas as pl

F = 10000
H = 64
BF = 640
NB = -(-F // BF)
FPAD = NB * BF


def _body(m_ref, wc_ref, ht_ref, x_ref, wx_ref, bias_ref,
          w1_ref, b1_ref, w2_ref, b2_ref, accz_ref, accm_ref, out_ref):
    k = pl.program_id(0)

    @pl.when(k == 0)
    def _():
        accz_ref[...] = jnp.zeros_like(accz_ref)
        accm_ref[...] = jnp.zeros_like(accm_ref)

    m = m_ref[...]
    wc = wc_ref[...]
    ht = ht_ref[...]
    h = jnp.sum(wc * ht[None, :, :], axis=1)
    z = x_ref[...] * wx_ref[...] + bias_ref[...] + h
    zs = jnp.where(m > 0.0, z, 0.0)
    accz_ref[...] += m * jnp.tanh(zs)
    accm_ref[...] += jnp.broadcast_to(m, (8, BF))

    @pl.when(k == NB - 1)
    def _():
        s = jnp.sum(accz_ref[...], axis=1)
        cnt = jnp.sum(accm_ref[0:1, :])
        iv = (s / jnp.maximum(cnt, 1.0)).reshape(1, H)
        hmlp = jnp.maximum(
            jax.lax.dot_general(iv, w1_ref[...],
                                (((1,), (1,)), ((), ()))) + b1_ref[...],
            0.0)
        logits = jax.lax.dot_general(hmlp, w2_ref[...],
                                     (((1,), (1,)), ((), ()))) + b2_ref[...]
        mx = jnp.max(logits, axis=1, keepdims=True)
        e = jnp.exp(logits - mx)
        p = e / jnp.sum(e, axis=1, keepdims=True)
        pad = jnp.concatenate([p, jnp.zeros((1, 126), jnp.float32)], axis=1)
        out_ref[...] = jnp.broadcast_to(pad, (8, 128))


@jax.jit
def _run(X, mask, Ht, Wx, Wc, rnn_bias, W1, b1, W2, b2):
    wcT = jnp.transpose(Wc, (1, 2, 0))
    htT = Ht.T
    wxT = Wx[:, :, 0].T
    biasT = rnn_bias.T
    xb = X.reshape(1, F)
    maskp = jnp.pad(mask.astype(jnp.float32), (0, FPAD - F)).reshape(1, FPAD)

    _, _, res = pl.pallas_call(
        _body,
        grid=(NB,),
        in_specs=[
            pl.BlockSpec((1, BF), lambda k: (0, k)),
            pl.BlockSpec((H, H, BF), lambda k: (0, 0, k)),
            pl.BlockSpec((H, BF), lambda k: (0, k)),
            pl.BlockSpec((1, BF), lambda k: (0, k)),
            pl.BlockSpec((H, BF), lambda k: (0, k)),
            pl.BlockSpec((H, BF), lambda k: (0, k)),
            pl.BlockSpec((H, H), lambda k: (0, 0)),
            pl.BlockSpec((1, H), lambda k: (0, 0)),
            pl.BlockSpec((2, H), lambda k: (0, 0)),
            pl.BlockSpec((1, 2), lambda k: (0, 0)),
        ],
        out_specs=[
            pl.BlockSpec((H, BF), lambda k: (0, 0)),
            pl.BlockSpec((8, BF), lambda k: (0, 0)),
            pl.BlockSpec((8, 128), lambda k: (0, 0)),
        ],
        out_shape=[
            jax.ShapeDtypeStruct((H, BF), jnp.float32),
            jax.ShapeDtypeStruct((8, BF), jnp.float32),
            jax.ShapeDtypeStruct((8, 128), jnp.float32),
        ],
    )(maskp, wcT, htT, xb, wxT, biasT,
      W1, b1.reshape(1, H), W2, b2.reshape(1, 2))
    return res[0, :2]


def kernel(tim, X, X_hap, mask, Ht, Wx, Wc, rnn_bias, W1, b1, W2, b2):
    pred = _run(X, mask, Ht, Wx, Wc, rnn_bias, W1, b1, W2, b2)
    H_curr = jnp.zeros((F, H), dtype=jnp.float32)
    return pred, H_curr
